# baseline (speedup 1.0000x reference)
import functools

import jax
import jax.numpy as jnp
from jax import lax
from jax.experimental import pallas as pl
from jax.experimental.pallas import tpu as pltpu
from jax.experimental.pallas import tpu_sc as plsc

N = 50000
E = 1600000
F = 1433
HID = 16
OUTW = 7

NTILES = 32
CHUNK = 128
KT = 394
E_PAD = NTILES * CHUNK * KT
N_PAD = 50176
RPT = N_PAD // 16
ZR = 196
WD = 16

_MESH = plsc.VectorSubcoreMesh(core_axis_name="c", subcore_axis_name="s")
_SC_PARAMS = pltpu.CompilerParams(use_tc_tiling_on_sc=False,
                                  needs_layout_passes=False)
_SC_PARAMS_NLP = _SC_PARAMS


def _zero_fill(zb, w):
    def fz(i, carry):
        zb[i, :] = jnp.zeros((w,), jnp.float32)
        return carry
    lax.fori_loop(0, ZR, fz, 0)


@functools.partial(
    pl.kernel,
    out_type=jax.ShapeDtypeStruct((2, 2, N_PAD, WD), jnp.float32),
    mesh=_MESH,
    scratch_types=[
        pltpu.VMEM((2, CHUNK), jnp.int32),
        pltpu.VMEM((N_PAD,), jnp.float32),
        pltpu.VMEM((RPT,), jnp.float32),
        pltpu.VMEM((RPT,), jnp.float32),
        pltpu.VMEM((ZR, WD), jnp.float32),
        pltpu.VMEM_SHARED((16, N_PAD), jnp.float32),
        pltpu.SemaphoreType.DMA,
        pltpu.SemaphoreType.DMA,
    ],
    compiler_params=_SC_PARAMS_NLP,
)
def _deg(src_hbm, dst_hbm, out_hbm, idx_v, hist, rbuf, tbuf, obuf, stage_sh,
         si0, si1):
    c = lax.axis_index("c")
    s = lax.axis_index("s")
    wid = c * 16 + s
    si = (si0, si1)
    ones16 = jnp.ones((16,), jnp.float32)

    for phase, ind_hbm in enumerate((src_hbm, dst_hbm)):
        def hz(i, carry):
            hist[pl.ds(i * 16, 16)] = jnp.zeros((16,), jnp.float32)
            return carry
        lax.fori_loop(0, N_PAD // 16, hz, 0)

        pltpu.async_copy(ind_hbm.at[wid, 0], idx_v.at[0], si0)
        pltpu.async_copy(ind_hbm.at[wid, 1], idx_v.at[1], si1)

        def scat(b, j):
            for k in range(CHUNK // 16):
                idx = idx_v[b, pl.ds(k * 16, 16)]
                plsc.addupdate_scatter(hist, [idx], ones16)

        def pair(jp, carry):
            j2 = jp * 2
            for b in (0, 1):
                j = j2 + b
                pltpu.make_async_copy(ind_hbm.at[wid, j], idx_v.at[b],
                                      si[b]).wait()
                scat(b, j)
                pltpu.async_copy(ind_hbm.at[wid, j + 2], idx_v.at[b], si[b])
            return carry
        lax.fori_loop(0, (KT - 2) // 2, pair, 0)
        for b in (0, 1):
            j = KT - 2 + b
            pltpu.make_async_copy(ind_hbm.at[wid, j], idx_v.at[b],
                                  si[b]).wait()
            scat(b, j)

        pltpu.sync_copy(hist, stage_sh.at[s])
        plsc.subcore_barrier()
        pltpu.sync_copy(stage_sh.at[0, pl.ds(s * RPT, RPT)], rbuf)

        def red(p, carry):
            pltpu.sync_copy(stage_sh.at[p, pl.ds(s * RPT, RPT)], tbuf)

            def add16(i, carry2):
                sl = pl.ds(i * 16, 16)
                rbuf[sl] = rbuf[sl] + tbuf[sl]
                return carry2
            lax.fori_loop(0, RPT // 16, add16, 0)
            return carry
        lax.fori_loop(1, 16, red, 0)

        for g in range(RPT // ZR):
            def rep(r, carry):
                iv = jnp.full((16,), r, dtype=jnp.int32) + g * ZR
                obuf[r, :] = plsc.load_gather(rbuf, [iv])
                return carry
            lax.fori_loop(0, ZR, rep, 0)
            pltpu.sync_copy(obuf,
                            out_hbm.at[c, phase, pl.ds(s * RPT + g * ZR, ZR)])
        plsc.subcore_barrier()


@functools.partial(
    pl.kernel,
    out_type=jax.ShapeDtypeStruct((2, N_PAD, 16), jnp.float32),
    name="prop",
    mesh=_MESH,
    scratch_types=[
        pltpu.VMEM((KT, CHUNK), jnp.int32),
        pltpu.VMEM((2, CHUNK), jnp.int32),
        pltpu.VMEM((2, CHUNK, 16), jnp.float32),
        pltpu.VMEM((ZR, 16), jnp.float32),
        pltpu.VMEM_SHARED((N_PAD, 16), jnp.float32),
        pltpu.SemaphoreType.DMA,
        pltpu.SemaphoreType.DMA,
        pltpu.SemaphoreType.DMA,
        pltpu.SemaphoreType.DMA,
    ],
    compiler_params=_SC_PARAMS,
)
def _prop(h_hbm, src_hbm, dst_hbm, out_hbm, src_v, dstb, rows, zb, acc_sh,
          sg0, sg1, sd0, sd1):
    c = lax.axis_index("c")
    s = lax.axis_index("s")
    wid = c * 16 + s
    sg = (sg0, sg1)
    sd = (sd0, sd1)
    _zero_fill(zb, 16)

    def zcp(i, carry):
        pltpu.sync_copy(zb, acc_sh.at[pl.ds(s * RPT + i * ZR, ZR)])
        return carry
    lax.fori_loop(0, RPT // ZR, zcp, 0)
    plsc.subcore_barrier()

    pltpu.sync_copy(src_hbm.at[wid], src_v)
    pltpu.async_copy(dst_hbm.at[wid, 0], dstb.at[0], sd0)
    pltpu.async_copy(dst_hbm.at[wid, 1], dstb.at[1], sd1)
    pltpu.async_copy(h_hbm.at[src_v.at[0]], rows.at[0], sg0)

    def pair(jp, carry):
        j2 = jp * 2
        for b in (0, 1):
            j = j2 + b
            pltpu.make_async_copy(h_hbm.at[src_v.at[j]], rows.at[b],
                                  sg[b]).wait()
            pltpu.async_copy(h_hbm.at[src_v.at[j + 1]], rows.at[1 - b],
                             sg[1 - b])
            pltpu.make_async_copy(dst_hbm.at[wid, j], dstb.at[b],
                                  sd[b]).wait()
            pltpu.sync_copy(rows.at[b], acc_sh.at[dstb.at[b]], add=True)
            pltpu.async_copy(dst_hbm.at[wid, j + 2], dstb.at[b], sd[b])
        return carry
    lax.fori_loop(0, (KT - 2) // 2, pair, 0)

    for b in (0, 1):
        j = KT - 2 + b
        pltpu.make_async_copy(h_hbm.at[src_v.at[j]], rows.at[b], sg[b]).wait()
        if b == 0:
            pltpu.async_copy(h_hbm.at[src_v.at[KT - 1]], rows.at[1], sg1)
        pltpu.make_async_copy(dst_hbm.at[wid, j], dstb.at[b], sd[b]).wait()
        pltpu.sync_copy(rows.at[b], acc_sh.at[dstb.at[b]], add=True)
    plsc.subcore_barrier()

    pltpu.sync_copy(acc_sh.at[pl.ds(s * RPT, RPT)],
                    out_hbm.at[c, pl.ds(s * RPT, RPT)])


BM1 = 2000
NW = N_PAD // 8
NWB = 224


def _mm1_body(x_ref, w_ref, h_ref):
    h_ref[...] = jnp.dot(x_ref[...], w_ref[...],
                         preferred_element_type=jnp.float32)


_mm1 = pl.pallas_call(
    _mm1_body,
    grid=(N // BM1,),
    in_specs=[
        pl.BlockSpec((BM1, F), lambda i: (i, 0)),
        pl.BlockSpec((F, HID), lambda i: (0, 0)),
    ],
    out_specs=pl.BlockSpec((BM1, HID), lambda i: (i, 0)),
    out_shape=jax.ShapeDtypeStruct((N, HID), jnp.float32),
)


def _donw(degw_ref):
    return lax.rsqrt(jnp.maximum(degw_ref[0] + degw_ref[2], 1.0))


def _dinw(degw_ref):
    return lax.rsqrt(jnp.maximum(degw_ref[1] + degw_ref[3], 1.0))


def _sc1_body(hw_ref, degw_ref, o_ref):
    o_ref[...] = hw_ref[...] * _donw(degw_ref)


_sc1 = pl.pallas_call(
    _sc1_body,
    grid=(NW // NWB,),
    in_specs=[
        pl.BlockSpec((NWB, 128), lambda i: (i, 0)),
        pl.BlockSpec((4, NWB, 128), lambda i: (0, i, 0)),
    ],
    out_specs=pl.BlockSpec((NWB, 128), lambda i: (i, 0)),
    out_shape=jax.ShapeDtypeStruct((NW, 128), jnp.float32),
)


def _mm2_body(p1_ref, degw_ref, w2_ref, b1_ref, h2_ref):
    h1 = jnp.maximum((p1_ref[0] + p1_ref[1]) * _dinw(degw_ref) + b1_ref[...],
                     0.0)
    h2 = jnp.dot(h1, w2_ref[...], preferred_element_type=jnp.float32)
    h2_ref[...] = h2 * _donw(degw_ref)


_mm2 = pl.pallas_call(
    _mm2_body,
    grid=(NW // NWB,),
    in_specs=[
        pl.BlockSpec((2, NWB, 128), lambda i: (0, i, 0)),
        pl.BlockSpec((4, NWB, 128), lambda i: (0, i, 0)),
        pl.BlockSpec((128, 128), lambda i: (0, 0)),
        pl.BlockSpec((1, 128), lambda i: (0, 0)),
    ],
    out_specs=pl.BlockSpec((NWB, 128), lambda i: (i, 0)),
    out_shape=jax.ShapeDtypeStruct((NW, 128), jnp.float32),
)


def _fin_body(p2_ref, degw_ref, b2_ref, o_ref):
    o_ref[...] = (p2_ref[0] + p2_ref[1]) * _dinw(degw_ref) + b2_ref[...]


_fin = pl.pallas_call(
    _fin_body,
    grid=(NW // NWB,),
    in_specs=[
        pl.BlockSpec((2, NWB, 128), lambda i: (0, i, 0)),
        pl.BlockSpec((4, NWB, 128), lambda i: (0, i, 0)),
        pl.BlockSpec((1, 128), lambda i: (0, 0)),
    ],
    out_specs=pl.BlockSpec((NWB, 128), lambda i: (i, 0)),
    out_shape=jax.ShapeDtypeStruct((NW, 128), jnp.float32),
)


def kernel(features_, edge_index, W1, b1, W2, b2):
    src = edge_index[0].astype(jnp.int32)
    dst = edge_index[1].astype(jnp.int32)
    npe = E_PAD - E
    pad_g = jnp.arange(npe, dtype=jnp.int32) % 8192
    pad_d = N + jnp.arange(npe, dtype=jnp.int32) % (N_PAD - N)
    srcp = jnp.concatenate([src, pad_g]).reshape(NTILES, KT, CHUNK)
    dstp = jnp.concatenate([dst, pad_d]).reshape(NTILES, KT, CHUNK)
    srcd = jnp.concatenate([src, pad_d]).reshape(NTILES, KT, CHUNK)
    dstd = jnp.concatenate([dst, pad_d]).reshape(NTILES, KT, CHUNK)

    w2p = jnp.pad(W2, ((0, 0), (0, HID - OUTW)))
    w2bd = jnp.kron(jnp.eye(8, dtype=jnp.float32), w2p)
    b1t = jnp.tile(b1, 8).reshape(1, 128)
    b2t = jnp.tile(jnp.pad(b2, (0, HID - OUTW)), 8).reshape(1, 128)

    degp = _deg(srcd, dstd)
    degw = degp.reshape(4, NW, 128)
    h1r = _mm1(features_, W1)
    h1sw = _sc1(h1r.reshape(N // 8, 128), degw)
    p1 = _prop(h1sw.reshape(N_PAD, HID), srcp, dstp)
    h2w = _mm2(p1.reshape(2, NW, 128), degw, w2bd, b1t)
    p2 = _prop(h2w.reshape(N_PAD, HID), srcp, dstp)
    outw = _fin(p2.reshape(2, NW, 128), degw, b2t)
    return outw.reshape(N_PAD, HID)[:N, :OUTW]


# speedup vs baseline: 1.0189x; 1.0189x over previous
import functools

import jax
import jax.numpy as jnp
from jax import lax
from jax.experimental import pallas as pl
from jax.experimental.pallas import tpu as pltpu
from jax.experimental.pallas import tpu_sc as plsc

N = 50000
E = 1600000
F = 1433
HID = 16
OUTW = 7

NTILES = 32
CHUNK = 128
KT = 394
E_PAD = NTILES * CHUNK * KT
N_PAD = 50176
RPT = N_PAD // 16
ZR = 196
WD = 16

_MESH = plsc.VectorSubcoreMesh(core_axis_name="c", subcore_axis_name="s")
_SC_PARAMS = pltpu.CompilerParams(use_tc_tiling_on_sc=False,
                                  needs_layout_passes=False)
_SC_PARAMS_NLP = _SC_PARAMS


def _zero_fill(zb, w):
    def fz(i, carry):
        zb[i, :] = jnp.zeros((w,), jnp.float32)
        return carry
    lax.fori_loop(0, ZR, fz, 0)


@functools.partial(
    pl.kernel,
    out_type=jax.ShapeDtypeStruct((2, 2, N_PAD, WD), jnp.float32),
    mesh=_MESH,
    scratch_types=[
        pltpu.VMEM((2, CHUNK), jnp.int32),
        pltpu.VMEM((N_PAD,), jnp.float32),
        pltpu.VMEM((RPT,), jnp.float32),
        pltpu.VMEM((RPT,), jnp.float32),
        pltpu.VMEM((ZR, WD), jnp.float32),
        pltpu.VMEM_SHARED((16, N_PAD), jnp.float32),
        pltpu.SemaphoreType.DMA,
        pltpu.SemaphoreType.DMA,
    ],
    compiler_params=_SC_PARAMS_NLP,
)
def _deg(src_hbm, dst_hbm, out_hbm, idx_v, hist, rbuf, tbuf, obuf, stage_sh,
         si0, si1):
    c = lax.axis_index("c")
    s = lax.axis_index("s")
    wid = c * 16 + s
    si = (si0, si1)
    ones16 = jnp.ones((16,), jnp.float32)

    for phase, ind_hbm in enumerate((src_hbm, dst_hbm)):
        def hz(i, carry):
            for u in range(8):
                hist[pl.ds(i * 128 + u * 16, 16)] = jnp.zeros((16,),
                                                              jnp.float32)
            return carry
        lax.fori_loop(0, N_PAD // 128, hz, 0)

        pltpu.async_copy(ind_hbm.at[wid, 0], idx_v.at[0], si0)
        pltpu.async_copy(ind_hbm.at[wid, 1], idx_v.at[1], si1)

        def scat(b, j):
            for k in range(CHUNK // 16):
                idx = idx_v[b, pl.ds(k * 16, 16)]
                plsc.addupdate_scatter(hist, [idx], ones16)

        def pair(jp, carry):
            j2 = jp * 2
            for b in (0, 1):
                j = j2 + b
                pltpu.make_async_copy(ind_hbm.at[wid, j], idx_v.at[b],
                                      si[b]).wait()
                scat(b, j)
                pltpu.async_copy(ind_hbm.at[wid, j + 2], idx_v.at[b], si[b])
            return carry
        lax.fori_loop(0, (KT - 2) // 2, pair, 0)
        for b in (0, 1):
            j = KT - 2 + b
            pltpu.make_async_copy(ind_hbm.at[wid, j], idx_v.at[b],
                                  si[b]).wait()
            scat(b, j)

        pltpu.sync_copy(hist, stage_sh.at[s])
        plsc.subcore_barrier()
        pltpu.sync_copy(stage_sh.at[0, pl.ds(s * RPT, RPT)], rbuf)

        def red(p, carry):
            pltpu.sync_copy(stage_sh.at[p, pl.ds(s * RPT, RPT)], tbuf)

            def add16(i, carry2):
                for u in range(4):
                    sl = pl.ds(i * 64 + u * 16, 16)
                    rbuf[sl] = rbuf[sl] + tbuf[sl]
                return carry2
            lax.fori_loop(0, RPT // 64, add16, 0)
            return carry
        lax.fori_loop(1, 16, red, 0)

        for g in range(RPT // ZR):
            def rep(r4, carry):
                for u in range(4):
                    r = r4 * 4 + u
                    iv = jnp.full((16,), r, dtype=jnp.int32) + g * ZR
                    obuf[r, :] = plsc.load_gather(rbuf, [iv])
                return carry
            lax.fori_loop(0, ZR // 4, rep, 0)
            pltpu.sync_copy(obuf,
                            out_hbm.at[c, phase, pl.ds(s * RPT + g * ZR, ZR)])
        plsc.subcore_barrier()


@functools.partial(
    pl.kernel,
    out_type=jax.ShapeDtypeStruct((2, N_PAD, 16), jnp.float32),
    name="prop",
    mesh=_MESH,
    scratch_types=[
        pltpu.VMEM((KT, CHUNK), jnp.int32),
        pltpu.VMEM((2, CHUNK), jnp.int32),
        pltpu.VMEM((2, CHUNK, 16), jnp.float32),
        pltpu.VMEM((ZR, 16), jnp.float32),
        pltpu.VMEM_SHARED((N_PAD, 16), jnp.float32),
        pltpu.SemaphoreType.DMA,
        pltpu.SemaphoreType.DMA,
        pltpu.SemaphoreType.DMA,
        pltpu.SemaphoreType.DMA,
    ],
    compiler_params=_SC_PARAMS,
)
def _prop(h_hbm, src_hbm, dst_hbm, out_hbm, src_v, dstb, rows, zb, acc_sh,
          sg0, sg1, sd0, sd1):
    c = lax.axis_index("c")
    s = lax.axis_index("s")
    wid = c * 16 + s
    sg = (sg0, sg1)
    sd = (sd0, sd1)
    _zero_fill(zb, 16)

    def zcp(i, carry):
        pltpu.sync_copy(zb, acc_sh.at[pl.ds(s * RPT + i * ZR, ZR)])
        return carry
    lax.fori_loop(0, RPT // ZR, zcp, 0)
    plsc.subcore_barrier()

    pltpu.sync_copy(src_hbm.at[wid], src_v)
    pltpu.async_copy(dst_hbm.at[wid, 0], dstb.at[0], sd0)
    pltpu.async_copy(dst_hbm.at[wid, 1], dstb.at[1], sd1)
    pltpu.async_copy(h_hbm.at[src_v.at[0]], rows.at[0], sg0)

    def pair(jp, carry):
        j2 = jp * 2
        for b in (0, 1):
            j = j2 + b
            pltpu.make_async_copy(h_hbm.at[src_v.at[j]], rows.at[b],
                                  sg[b]).wait()
            pltpu.async_copy(h_hbm.at[src_v.at[j + 1]], rows.at[1 - b],
                             sg[1 - b])
            pltpu.make_async_copy(dst_hbm.at[wid, j], dstb.at[b],
                                  sd[b]).wait()
            pltpu.sync_copy(rows.at[b], acc_sh.at[dstb.at[b]], add=True)
            pltpu.async_copy(dst_hbm.at[wid, j + 2], dstb.at[b], sd[b])
        return carry
    lax.fori_loop(0, (KT - 2) // 2, pair, 0)

    for b in (0, 1):
        j = KT - 2 + b
        pltpu.make_async_copy(h_hbm.at[src_v.at[j]], rows.at[b], sg[b]).wait()
        if b == 0:
            pltpu.async_copy(h_hbm.at[src_v.at[KT - 1]], rows.at[1], sg1)
        pltpu.make_async_copy(dst_hbm.at[wid, j], dstb.at[b], sd[b]).wait()
        pltpu.sync_copy(rows.at[b], acc_sh.at[dstb.at[b]], add=True)
    plsc.subcore_barrier()

    pltpu.sync_copy(acc_sh.at[pl.ds(s * RPT, RPT)],
                    out_hbm.at[c, pl.ds(s * RPT, RPT)])


BM1 = 2000
NW = N_PAD // 8
NWB = 224


def _mm1_body(x_ref, w_ref, h_ref):
    h_ref[...] = jnp.dot(x_ref[...], w_ref[...],
                         preferred_element_type=jnp.float32)


_mm1 = pl.pallas_call(
    _mm1_body,
    grid=(N // BM1,),
    in_specs=[
        pl.BlockSpec((BM1, F), lambda i: (i, 0)),
        pl.BlockSpec((F, HID), lambda i: (0, 0)),
    ],
    out_specs=pl.BlockSpec((BM1, HID), lambda i: (i, 0)),
    out_shape=jax.ShapeDtypeStruct((N, HID), jnp.float32),
)


def _donw(degw_ref):
    return lax.rsqrt(jnp.maximum(degw_ref[0] + degw_ref[2], 1.0))


def _dinw(degw_ref):
    return lax.rsqrt(jnp.maximum(degw_ref[1] + degw_ref[3], 1.0))


def _sc1_body(hw_ref, degw_ref, o_ref):
    o_ref[...] = hw_ref[...] * _donw(degw_ref)


_sc1 = pl.pallas_call(
    _sc1_body,
    grid=(NW // NWB,),
    in_specs=[
        pl.BlockSpec((NWB, 128), lambda i: (i, 0)),
        pl.BlockSpec((4, NWB, 128), lambda i: (0, i, 0)),
    ],
    out_specs=pl.BlockSpec((NWB, 128), lambda i: (i, 0)),
    out_shape=jax.ShapeDtypeStruct((NW, 128), jnp.float32),
)


def _mm2_body(p1_ref, degw_ref, w2_ref, b1_ref, h2_ref):
    h1 = jnp.maximum((p1_ref[0] + p1_ref[1]) * _dinw(degw_ref) + b1_ref[...],
                     0.0)
    h2 = jnp.dot(h1, w2_ref[...], preferred_element_type=jnp.float32)
    h2_ref[...] = h2 * _donw(degw_ref)


_mm2 = pl.pallas_call(
    _mm2_body,
    grid=(NW // NWB,),
    in_specs=[
        pl.BlockSpec((2, NWB, 128), lambda i: (0, i, 0)),
        pl.BlockSpec((4, NWB, 128), lambda i: (0, i, 0)),
        pl.BlockSpec((128, 128), lambda i: (0, 0)),
        pl.BlockSpec((1, 128), lambda i: (0, 0)),
    ],
    out_specs=pl.BlockSpec((NWB, 128), lambda i: (i, 0)),
    out_shape=jax.ShapeDtypeStruct((NW, 128), jnp.float32),
)


def _fin_body(p2_ref, degw_ref, b2_ref, o_ref):
    o_ref[...] = (p2_ref[0] + p2_ref[1]) * _dinw(degw_ref) + b2_ref[...]


_fin = pl.pallas_call(
    _fin_body,
    grid=(NW // NWB,),
    in_specs=[
        pl.BlockSpec((2, NWB, 128), lambda i: (0, i, 0)),
        pl.BlockSpec((4, NWB, 128), lambda i: (0, i, 0)),
        pl.BlockSpec((1, 128), lambda i: (0, 0)),
    ],
    out_specs=pl.BlockSpec((NWB, 128), lambda i: (i, 0)),
    out_shape=jax.ShapeDtypeStruct((NW, 128), jnp.float32),
)


def kernel(features_, edge_index, W1, b1, W2, b2):
    src = edge_index[0].astype(jnp.int32)
    dst = edge_index[1].astype(jnp.int32)
    npe = E_PAD - E
    pad_g = jnp.arange(npe, dtype=jnp.int32) % 8192
    pad_d = N + jnp.arange(npe, dtype=jnp.int32) % (N_PAD - N)
    srcp = jnp.concatenate([src, pad_g]).reshape(NTILES, KT, CHUNK)
    dstp = jnp.concatenate([dst, pad_d]).reshape(NTILES, KT, CHUNK)
    srcd = jnp.concatenate([src, pad_d]).reshape(NTILES, KT, CHUNK)
    dstd = jnp.concatenate([dst, pad_d]).reshape(NTILES, KT, CHUNK)

    w2p = jnp.pad(W2, ((0, 0), (0, HID - OUTW)))
    w2bd = jnp.kron(jnp.eye(8, dtype=jnp.float32), w2p)
    b1t = jnp.tile(b1, 8).reshape(1, 128)
    b2t = jnp.tile(jnp.pad(b2, (0, HID - OUTW)), 8).reshape(1, 128)

    degp = _deg(srcd, dstd)
    degw = degp.reshape(4, NW, 128)
    h1r = _mm1(features_, W1)
    h1sw = _sc1(h1r.reshape(N // 8, 128), degw)
    p1 = _prop(h1sw.reshape(N_PAD, HID), srcp, dstp)
    h2w = _mm2(p1.reshape(2, NW, 128), degw, w2bd, b1t)
    p2 = _prop(h2w.reshape(N_PAD, HID), srcp, dstp)
    outw = _fin(p2.reshape(2, NW, 128), degw, b2t)
    return outw.reshape(N_PAD, HID)[:N, :OUTW]
